# baseline (speedup 1.0000x reference)
import jax
import jax.numpy as jnp
from jax import lax
from jax.experimental import pallas as pl
from jax.experimental.pallas import tpu as pltpu
from jax.experimental.pallas import tpu_sc as plsc

E = 64
H = 768
I = 3072
S = 8192
T = 64
G = S // T + E
PAD = G * T
NC, NS, L = 2, 16, 16
NW = NC * NS
CHUNK = S // NW
BT = 512


def _gate_body(x_ref, wg_ref, idx_ref, p_ref, hist_ref):
    logits = jnp.dot(x_ref[...], wg_ref[...], preferred_element_type=jnp.float32)
    m = jnp.max(logits, axis=-1, keepdims=True)
    col = lax.broadcasted_iota(jnp.int32, logits.shape, 1)
    idx2 = jnp.min(jnp.where(logits >= m, col, E), axis=-1, keepdims=True)
    denom = jnp.sum(jnp.exp(logits - m), axis=-1)
    idx_ref[0, 0, :] = idx2[:, 0]
    p_ref[0, 0, :] = 1.0 / denom
    onehot = jnp.where(col == idx2, 1, 0)
    for c in range(BT // CHUNK):
        hist_ref[0, c, :] = jnp.sum(onehot[c * CHUNK:(c + 1) * CHUNK], axis=0)


def _gate(flat, wg):
    nb = S // BT
    return pl.pallas_call(
        _gate_body,
        grid=(nb,),
        in_specs=[
            pl.BlockSpec((BT, H), lambda b: (b, 0)),
            pl.BlockSpec((H, E), lambda b: (0, 0)),
        ],
        out_specs=[
            pl.BlockSpec((1, 1, BT), lambda b: (b, 0, 0)),
            pl.BlockSpec((1, 1, BT), lambda b: (b, 0, 0)),
            pl.BlockSpec((1, BT // CHUNK, E), lambda b: (b, 0, 0)),
        ],
        out_shape=[
            jax.ShapeDtypeStruct((nb, 1, BT), jnp.int32),
            jax.ShapeDtypeStruct((nb, 1, BT), jnp.float32),
            jax.ShapeDtypeStruct((nb, BT // CHUNK, E), jnp.int32),
        ],
    )(flat, wg)


def _route_body(idx_hbm, x_hbm, hist_hbm,
                counts_hbm, dpos_hbm, xs_hbm, bal_hbm,
                hist16, histw, allh, base, idxv, dposv,
                xbuf0, xbuf1, balw, rsem, wsem):
    cid = lax.axis_index("c")
    sid = lax.axis_index("s")
    wid = sid * NC + cid
    tbase = wid * CHUNK

    pltpu.sync_copy(idx_hbm.at[pl.ds(tbase, CHUNK)], idxv)
    pltpu.sync_copy(hist_hbm, allh)

    zero16 = jnp.zeros((L,), jnp.int32)
    for r in range(L):
        for c in range(E // L):
            hist16[pl.ds(r * E + c * L, L)] = zero16

    lanes = lax.iota(jnp.int32, L)
    laneoff = lanes * E
    for g in range(CHUNK // L):
        ev = idxv[pl.ds(g * L, L)]
        fl = laneoff + ev
        cur = plsc.load_gather(hist16, [fl])
        plsc.store_scatter(hist16, [fl], cur + 1)

    tots = []
    for c in range(E // L):
        acc = zero16
        for w in range(NW):
            acc = acc + allh[pl.ds(w * E + c * L, L)]
        tots.append(acc)
    carry = jnp.int32(0)
    offs = []
    for c in range(E // L):
        tiles_c = jnp.right_shift(tots[c] + (T - 1), jnp.int32(T.bit_length() - 1))
        inc = plsc.cumsum(tiles_c)
        offs.append((inc - tiles_c + carry) * T)
        carry = carry + jnp.sum(tiles_c)

    for c in range(E // L):
        prew = zero16
        for w in range(NW):
            hv = allh[pl.ds(w * E + c * L, L)]
            prew = prew + jnp.where(jnp.int32(w) < wid, hv, zero16)
        bacc = offs[c] + prew
        for l in range(L):
            base[pl.ds(l * E + c * L, L)] = bacc
            bacc = bacc + hist16[pl.ds(l * E + c * L, L)]

    for g in range(CHUNK // L):
        ev = idxv[pl.ds(g * L, L)]
        fl = laneoff + ev
        pos = plsc.load_gather(base, [fl])
        plsc.store_scatter(base, [fl], pos + 1)
        dposv[g >> 2, pl.ds((g & 3) * L, L)] = pos

    pltpu.sync_copy(dposv, dpos_hbm.at[wid])

    bufs = (xbuf0, xbuf1)
    nchunk = CHUNK // 64
    r = pltpu.async_copy(x_hbm.at[pl.ds(tbase, 64)], xbuf0, rsem)
    s_prev = None
    for c in range(nchunk):
        cur = bufs[c % 2]
        r.wait()
        if c + 1 < nchunk:
            if s_prev is not None:
                s_prev.wait()
            r = pltpu.async_copy(
                x_hbm.at[pl.ds(tbase + (c + 1) * 64, 64)], bufs[(c + 1) % 2], rsem)
        s = pltpu.async_copy(cur, xs_hbm.at[dposv.at[c]], wsem)
        if c + 1 >= nchunk and s_prev is not None:
            s_prev.wait()
        s_prev = s
    s_prev.wait()

    @pl.when(wid == 0)
    def _():
        for c in range(E // L):
            histw[pl.ds(c * L, L)] = tots[c]
        pltpu.sync_copy(histw, counts_hbm)
        s = jnp.float32(0)
        for c in range(E // L):
            f = tots[c].astype(jnp.float32) * jnp.float32(1.0 / S)
            s = s + jnp.sum(f * f)
        balw[...] = jnp.zeros((L,), jnp.float32) + s * jnp.float32(E)
        pltpu.sync_copy(balw, bal_hbm)


def _route(idx_flat, flat, hist):
    return pl.kernel(
        _route_body,
        out_type=[
            jax.ShapeDtypeStruct((E,), jnp.int32),
            jax.ShapeDtypeStruct((NW, CHUNK // 64, 64), jnp.int32),
            jax.ShapeDtypeStruct((PAD, H), jnp.float32),
            jax.ShapeDtypeStruct((L,), jnp.float32),
        ],
        mesh=plsc.VectorSubcoreMesh(core_axis_name="c", subcore_axis_name="s",
                                    num_cores=NC, num_subcores=NS),
        scratch_types=[
            pltpu.VMEM((L * E,), jnp.int32),
            pltpu.VMEM((E,), jnp.int32),
            pltpu.VMEM((NW * E,), jnp.int32),
            pltpu.VMEM((L * E,), jnp.int32),
            pltpu.VMEM((CHUNK,), jnp.int32),
            pltpu.VMEM((CHUNK // 64, 64), jnp.int32),
            pltpu.VMEM((64, H), jnp.float32),
            pltpu.VMEM((64, H), jnp.float32),
            pltpu.VMEM((L,), jnp.float32),
            pltpu.SemaphoreType.DMA,
            pltpu.SemaphoreType.DMA,
        ],
        compiler_params=pltpu.CompilerParams(needs_layout_passes=False),
    )(idx_flat, flat, hist)


def _ffn_body(te_ref, rb_ref, tot_ref, x_ref, w1_ref, w2_ref, o_ref):
    g = pl.program_id(0)

    @pl.when(g < tot_ref[0])
    def _():
        h = jnp.dot(x_ref[...], w1_ref[0], preferred_element_type=jnp.float32)
        h = h * (1.0 / (1.0 + jnp.exp(-h)))
        o_ref[...] = jnp.dot(h, w2_ref[0], preferred_element_type=jnp.float32)


def _ffn(te, rb, tot, xs, W1, W2):
    grid_spec = pltpu.PrefetchScalarGridSpec(
        num_scalar_prefetch=3,
        grid=(G,),
        in_specs=[
            pl.BlockSpec((T, H), lambda g, te, rb, tot: (rb[g], 0)),
            pl.BlockSpec((1, H, I), lambda g, te, rb, tot: (te[g], 0, 0)),
            pl.BlockSpec((1, I, H), lambda g, te, rb, tot: (te[g], 0, 0)),
        ],
        out_specs=pl.BlockSpec((T, H), lambda g, te, rb, tot: (rb[g], 0)),
    )
    return pl.pallas_call(
        _ffn_body,
        grid_spec=grid_spec,
        out_shape=jax.ShapeDtypeStruct((PAD, H), jnp.float32),
    )(te, rb, tot, xs, W1, W2)


def _unsort_body(ys_hbm, dpos_hbm, p_hbm, y_hbm, dposv, pv, ybuf0, ybuf1,
                 gsem, wsem):
    cid = lax.axis_index("c")
    sid = lax.axis_index("s")
    wid = sid * NC + cid
    tbase = wid * CHUNK
    pltpu.sync_copy(dpos_hbm.at[wid], dposv)
    pltpu.sync_copy(p_hbm.at[pl.ds(tbase, CHUNK)], pv)
    bufs = (ybuf0, ybuf1)
    nchunk = CHUNK // 64
    zeroL = jnp.zeros((L,), jnp.int32)
    g = pltpu.async_copy(ys_hbm.at[dposv.at[0]], ybuf0, gsem)
    w_prev = None
    for c in range(nchunk):
        cur = bufs[c % 2]
        g.wait()
        if c + 1 < nchunk:
            if w_prev is not None:
                w_prev.wait()
            g = pltpu.async_copy(ys_hbm.at[dposv.at[c + 1]], bufs[(c + 1) % 2], gsem)

        def _scale_row(j, _, cur=cur, c=c):
            pj = plsc.load_gather(pv, [zeroL + (c * 64 + j)])
            for k in range(H // L):
                cur[j, pl.ds(k * L, L)] = cur[j, pl.ds(k * L, L)] * pj
            return _

        lax.fori_loop(0, 64, _scale_row, 0)

        w = pltpu.async_copy(cur, y_hbm.at[pl.ds(tbase + c * 64, 64)], wsem)
        if c + 1 >= nchunk and w_prev is not None:
            w_prev.wait()
        w_prev = w
    w_prev.wait()


def _unsort(ys, dpos, p_flat):
    return pl.kernel(
        _unsort_body,
        out_type=jax.ShapeDtypeStruct((S, H), jnp.float32),
        mesh=plsc.VectorSubcoreMesh(core_axis_name="c", subcore_axis_name="s",
                                    num_cores=NC, num_subcores=NS),
        scratch_types=[
            pltpu.VMEM((CHUNK // 64, 64), jnp.int32),
            pltpu.VMEM((CHUNK,), jnp.float32),
            pltpu.VMEM((64, H), jnp.float32),
            pltpu.VMEM((64, H), jnp.float32),
            pltpu.SemaphoreType.DMA,
            pltpu.SemaphoreType.DMA,
        ],
        compiler_params=pltpu.CompilerParams(needs_layout_passes=False),
    )(ys, dpos, p_flat)


def kernel(x, W_gate, W1, W2):
    flat = x.reshape(S, H)
    idx3, p3, hist = _gate(flat, W_gate)
    idx_flat = idx3.reshape(S)
    counts, dpos, xs, balv = _route(idx_flat, flat, hist.reshape(NW * E))

    tiles = (counts + T - 1) // T
    tb = jnp.cumsum(tiles)
    total = tb[E - 1]
    gid = jnp.arange(G, dtype=jnp.int32)
    te_raw = jnp.searchsorted(tb, gid, side="right").astype(jnp.int32)
    last = jnp.maximum(total - 1, 0)
    te = jnp.where(gid < total, jnp.minimum(te_raw, E - 1), te_raw[last])
    rb = jnp.where(gid < total, gid, last)

    ys = _ffn(te, rb, total.reshape(1), xs, W1, W2)
    yflat = _unsort(ys, dpos, p3.reshape(S))

    return (
        yflat.reshape(1, S, H),
        idx3.reshape(1, S, 1),
        p3.reshape(1, S, 1),
        balv[0],
        counts,
    )


# speedup vs baseline: 1.4542x; 1.4542x over previous
import jax
import jax.numpy as jnp
from jax import lax
from jax.experimental import pallas as pl
from jax.experimental.pallas import tpu as pltpu
from jax.experimental.pallas import tpu_sc as plsc

E = 64
H = 768
I = 3072
S = 8192
T = 256
G = S // T + E
PAD = G * T
NC, NS, L = 2, 16, 16
NW = NC * NS
CHUNK = S // NW
BT = 512


def _gate_body(x_ref, wg_ref, idx_ref, p_ref, hist_ref):
    logits = jnp.dot(x_ref[...], wg_ref[...], preferred_element_type=jnp.float32)
    m = jnp.max(logits, axis=-1, keepdims=True)
    col = lax.broadcasted_iota(jnp.int32, logits.shape, 1)
    idx2 = jnp.min(jnp.where(logits >= m, col, E), axis=-1, keepdims=True)
    denom = jnp.sum(jnp.exp(logits - m), axis=-1)
    idx_ref[0, 0, :] = idx2[:, 0]
    p_ref[0, 0, :] = 1.0 / denom
    onehot = jnp.where(col == idx2, 1, 0)
    for c in range(BT // CHUNK):
        hist_ref[0, c, :] = jnp.sum(onehot[c * CHUNK:(c + 1) * CHUNK], axis=0)


def _gate(flat, wg):
    nb = S // BT
    return pl.pallas_call(
        _gate_body,
        grid=(nb,),
        in_specs=[
            pl.BlockSpec((BT, H), lambda b: (b, 0)),
            pl.BlockSpec((H, E), lambda b: (0, 0)),
        ],
        out_specs=[
            pl.BlockSpec((1, 1, BT), lambda b: (b, 0, 0)),
            pl.BlockSpec((1, 1, BT), lambda b: (b, 0, 0)),
            pl.BlockSpec((1, BT // CHUNK, E), lambda b: (b, 0, 0)),
        ],
        out_shape=[
            jax.ShapeDtypeStruct((nb, 1, BT), jnp.int32),
            jax.ShapeDtypeStruct((nb, 1, BT), jnp.float32),
            jax.ShapeDtypeStruct((nb, BT // CHUNK, E), jnp.int32),
        ],
    )(flat, wg)


def _route_body(idx_hbm, x_hbm, hist_hbm,
                counts_hbm, dpos_hbm, xs_hbm, bal_hbm,
                hist16, histw, allh, base, idxv, dposv,
                xbuf0, xbuf1, balw, rsem, wsem):
    cid = lax.axis_index("c")
    sid = lax.axis_index("s")
    wid = sid * NC + cid
    tbase = wid * CHUNK

    pltpu.sync_copy(idx_hbm.at[pl.ds(tbase, CHUNK)], idxv)
    pltpu.sync_copy(hist_hbm, allh)

    zero16 = jnp.zeros((L,), jnp.int32)
    for r in range(L):
        for c in range(E // L):
            hist16[pl.ds(r * E + c * L, L)] = zero16

    lanes = lax.iota(jnp.int32, L)
    laneoff = lanes * E
    for g in range(CHUNK // L):
        ev = idxv[pl.ds(g * L, L)]
        fl = laneoff + ev
        cur = plsc.load_gather(hist16, [fl])
        plsc.store_scatter(hist16, [fl], cur + 1)

    tots = []
    for c in range(E // L):
        acc = zero16
        for w in range(NW):
            acc = acc + allh[pl.ds(w * E + c * L, L)]
        tots.append(acc)
    carry = jnp.int32(0)
    offs = []
    for c in range(E // L):
        tiles_c = jnp.right_shift(tots[c] + (T - 1), jnp.int32(T.bit_length() - 1))
        inc = plsc.cumsum(tiles_c)
        offs.append((inc - tiles_c + carry) * T)
        carry = carry + jnp.sum(tiles_c)

    for c in range(E // L):
        prew = zero16
        for w in range(NW):
            hv = allh[pl.ds(w * E + c * L, L)]
            prew = prew + jnp.where(jnp.int32(w) < wid, hv, zero16)
        bacc = offs[c] + prew
        for l in range(L):
            base[pl.ds(l * E + c * L, L)] = bacc
            bacc = bacc + hist16[pl.ds(l * E + c * L, L)]

    for g in range(CHUNK // L):
        ev = idxv[pl.ds(g * L, L)]
        fl = laneoff + ev
        pos = plsc.load_gather(base, [fl])
        plsc.store_scatter(base, [fl], pos + 1)
        dposv[g >> 2, pl.ds((g & 3) * L, L)] = pos

    pltpu.sync_copy(dposv, dpos_hbm.at[wid])

    bufs = (xbuf0, xbuf1)
    nchunk = CHUNK // 64
    r = pltpu.async_copy(x_hbm.at[pl.ds(tbase, 64)], xbuf0, rsem)
    s_prev = None
    for c in range(nchunk):
        cur = bufs[c % 2]
        r.wait()
        if c + 1 < nchunk:
            if s_prev is not None:
                s_prev.wait()
            r = pltpu.async_copy(
                x_hbm.at[pl.ds(tbase + (c + 1) * 64, 64)], bufs[(c + 1) % 2], rsem)
        s = pltpu.async_copy(cur, xs_hbm.at[dposv.at[c]], wsem)
        if c + 1 >= nchunk and s_prev is not None:
            s_prev.wait()
        s_prev = s
    s_prev.wait()

    @pl.when(wid == 0)
    def _():
        for c in range(E // L):
            histw[pl.ds(c * L, L)] = tots[c]
        pltpu.sync_copy(histw, counts_hbm)
        s = jnp.float32(0)
        for c in range(E // L):
            f = tots[c].astype(jnp.float32) * jnp.float32(1.0 / S)
            s = s + jnp.sum(f * f)
        balw[...] = jnp.zeros((L,), jnp.float32) + s * jnp.float32(E)
        pltpu.sync_copy(balw, bal_hbm)


def _route(idx_flat, flat, hist):
    return pl.kernel(
        _route_body,
        out_type=[
            jax.ShapeDtypeStruct((E,), jnp.int32),
            jax.ShapeDtypeStruct((NW, CHUNK // 64, 64), jnp.int32),
            jax.ShapeDtypeStruct((PAD, H), jnp.float32),
            jax.ShapeDtypeStruct((L,), jnp.float32),
        ],
        mesh=plsc.VectorSubcoreMesh(core_axis_name="c", subcore_axis_name="s",
                                    num_cores=NC, num_subcores=NS),
        scratch_types=[
            pltpu.VMEM((L * E,), jnp.int32),
            pltpu.VMEM((E,), jnp.int32),
            pltpu.VMEM((NW * E,), jnp.int32),
            pltpu.VMEM((L * E,), jnp.int32),
            pltpu.VMEM((CHUNK,), jnp.int32),
            pltpu.VMEM((CHUNK // 64, 64), jnp.int32),
            pltpu.VMEM((64, H), jnp.float32),
            pltpu.VMEM((64, H), jnp.float32),
            pltpu.VMEM((L,), jnp.float32),
            pltpu.SemaphoreType.DMA,
            pltpu.SemaphoreType.DMA,
        ],
        compiler_params=pltpu.CompilerParams(needs_layout_passes=False),
    )(idx_flat, flat, hist)


def _ffn_body(te_ref, rb_ref, tot_ref, x_ref, w1_ref, w2_ref, o_ref):
    g = pl.program_id(0)

    @pl.when(g < tot_ref[0])
    def _():
        h = jnp.dot(x_ref[...], w1_ref[0], preferred_element_type=jnp.float32)
        h = h * (1.0 / (1.0 + jnp.exp(-h)))
        o_ref[...] = jnp.dot(h, w2_ref[0], preferred_element_type=jnp.float32)


def _ffn(te, rb, tot, xs, W1, W2):
    grid_spec = pltpu.PrefetchScalarGridSpec(
        num_scalar_prefetch=3,
        grid=(G,),
        in_specs=[
            pl.BlockSpec((T, H), lambda g, te, rb, tot: (rb[g], 0)),
            pl.BlockSpec((1, H, I), lambda g, te, rb, tot: (te[g], 0, 0)),
            pl.BlockSpec((1, I, H), lambda g, te, rb, tot: (te[g], 0, 0)),
        ],
        out_specs=pl.BlockSpec((T, H), lambda g, te, rb, tot: (rb[g], 0)),
    )
    return pl.pallas_call(
        _ffn_body,
        grid_spec=grid_spec,
        out_shape=jax.ShapeDtypeStruct((PAD, H), jnp.float32),
    )(te, rb, tot, xs, W1, W2)


def _unsort_body(ys_hbm, dpos_hbm, p_hbm, y_hbm, dposv, pv, ybuf0, ybuf1,
                 gsem, wsem):
    cid = lax.axis_index("c")
    sid = lax.axis_index("s")
    wid = sid * NC + cid
    tbase = wid * CHUNK
    pltpu.sync_copy(dpos_hbm.at[wid], dposv)
    pltpu.sync_copy(p_hbm.at[pl.ds(tbase, CHUNK)], pv)
    bufs = (ybuf0, ybuf1)
    nchunk = CHUNK // 64
    zeroL = jnp.zeros((L,), jnp.int32)
    g = pltpu.async_copy(ys_hbm.at[dposv.at[0]], ybuf0, gsem)
    w_prev = None
    for c in range(nchunk):
        cur = bufs[c % 2]
        g.wait()
        if c + 1 < nchunk:
            if w_prev is not None:
                w_prev.wait()
            g = pltpu.async_copy(ys_hbm.at[dposv.at[c + 1]], bufs[(c + 1) % 2], gsem)

        def _scale_row(j, _, cur=cur, c=c):
            pj = plsc.load_gather(pv, [zeroL + (c * 64 + j)])
            for k in range(H // L):
                cur[j, pl.ds(k * L, L)] = cur[j, pl.ds(k * L, L)] * pj
            return _

        lax.fori_loop(0, 64, _scale_row, 0)

        w = pltpu.async_copy(cur, y_hbm.at[pl.ds(tbase + c * 64, 64)], wsem)
        if c + 1 >= nchunk and w_prev is not None:
            w_prev.wait()
        w_prev = w
    w_prev.wait()


def _unsort(ys, dpos, p_flat):
    return pl.kernel(
        _unsort_body,
        out_type=jax.ShapeDtypeStruct((S, H), jnp.float32),
        mesh=plsc.VectorSubcoreMesh(core_axis_name="c", subcore_axis_name="s",
                                    num_cores=NC, num_subcores=NS),
        scratch_types=[
            pltpu.VMEM((CHUNK // 64, 64), jnp.int32),
            pltpu.VMEM((CHUNK,), jnp.float32),
            pltpu.VMEM((64, H), jnp.float32),
            pltpu.VMEM((64, H), jnp.float32),
            pltpu.SemaphoreType.DMA,
            pltpu.SemaphoreType.DMA,
        ],
        compiler_params=pltpu.CompilerParams(needs_layout_passes=False),
    )(ys, dpos, p_flat)


def kernel(x, W_gate, W1, W2):
    flat = x.reshape(S, H)
    idx3, p3, hist = _gate(flat, W_gate)
    idx_flat = idx3.reshape(S)
    counts, dpos, xs, balv = _route(idx_flat, flat, hist.reshape(NW * E))

    tiles = (counts + T - 1) // T
    tb = jnp.cumsum(tiles)
    total = tb[E - 1]
    gid = jnp.arange(G, dtype=jnp.int32)
    te_raw = jnp.searchsorted(tb, gid, side="right").astype(jnp.int32)
    last = jnp.maximum(total - 1, 0)
    te = jnp.where(gid < total, jnp.minimum(te_raw, E - 1), te_raw[last])
    rb = jnp.where(gid < total, gid, last)

    ys = _ffn(te, rb, total.reshape(1), xs, W1, W2)
    yflat = _unsort(ys, dpos, p3.reshape(S))

    return (
        yflat.reshape(1, S, H),
        idx3.reshape(1, S, 1),
        p3.reshape(1, S, 1),
        balv[0],
        counts,
    )


# speedup vs baseline: 1.4636x; 1.0065x over previous
import jax
import jax.numpy as jnp
from jax import lax
from jax.experimental import pallas as pl
from jax.experimental.pallas import tpu as pltpu
from jax.experimental.pallas import tpu_sc as plsc

E = 64
H = 768
I = 3072
S = 8192
T = 256
G = S // T + E
PAD = G * T
NC, NS, L = 2, 16, 16
NW = NC * NS
CHUNK = S // NW
BT = 512


def _gate_body(x_ref, wg_ref, idx_ref, p_ref, hist_ref):
    logits = jnp.dot(x_ref[...], wg_ref[...], preferred_element_type=jnp.float32)
    m = jnp.max(logits, axis=-1, keepdims=True)
    col = lax.broadcasted_iota(jnp.int32, logits.shape, 1)
    idx2 = jnp.min(jnp.where(logits >= m, col, E), axis=-1, keepdims=True)
    denom = jnp.sum(jnp.exp(logits - m), axis=-1)
    idx_ref[0, 0, :] = idx2[:, 0]
    p_ref[0, 0, :] = 1.0 / denom
    onehot = jnp.where(col == idx2, 1, 0)
    for c in range(BT // CHUNK):
        hist_ref[0, c, :] = jnp.sum(onehot[c * CHUNK:(c + 1) * CHUNK], axis=0)


def _gate(flat, wg):
    nb = S // BT
    return pl.pallas_call(
        _gate_body,
        grid=(nb,),
        in_specs=[
            pl.BlockSpec((BT, H), lambda b: (b, 0)),
            pl.BlockSpec((H, E), lambda b: (0, 0)),
        ],
        out_specs=[
            pl.BlockSpec((1, 1, BT), lambda b: (b, 0, 0)),
            pl.BlockSpec((1, 1, BT), lambda b: (b, 0, 0)),
            pl.BlockSpec((1, BT // CHUNK, E), lambda b: (b, 0, 0)),
        ],
        out_shape=[
            jax.ShapeDtypeStruct((nb, 1, BT), jnp.int32),
            jax.ShapeDtypeStruct((nb, 1, BT), jnp.float32),
            jax.ShapeDtypeStruct((nb, BT // CHUNK, E), jnp.int32),
        ],
    )(flat, wg)


def _route_body(idx_hbm, x_hbm, hist_hbm,
                counts_hbm, dpos_hbm, xs_hbm, bal_hbm,
                hist16, histw, allh, base, idxv, dposv,
                xbuf0, xbuf1, balw, rsem, wsem):
    cid = lax.axis_index("c")
    sid = lax.axis_index("s")
    wid = sid * NC + cid
    tbase = wid * CHUNK

    pltpu.sync_copy(idx_hbm.at[pl.ds(tbase, CHUNK)], idxv)
    pltpu.sync_copy(hist_hbm, allh)

    r0 = pltpu.async_copy(x_hbm.at[pl.ds(tbase, 64)], xbuf0, rsem)
    r1 = pltpu.async_copy(x_hbm.at[pl.ds(tbase + 64, 64)], xbuf1, rsem)

    zero16 = jnp.zeros((L,), jnp.int32)
    for r in range(L):
        for c in range(E // L):
            hist16[pl.ds(r * E + c * L, L)] = zero16

    lanes = lax.iota(jnp.int32, L)
    laneoff = lanes * E
    for g in range(CHUNK // L):
        ev = idxv[pl.ds(g * L, L)]
        fl = laneoff + ev
        cur = plsc.load_gather(hist16, [fl])
        plsc.store_scatter(hist16, [fl], cur + 1)

    tots = []
    for c in range(E // L):
        acc = zero16
        for w in range(NW):
            acc = acc + allh[pl.ds(w * E + c * L, L)]
        tots.append(acc)
    carry = jnp.int32(0)
    offs = []
    for c in range(E // L):
        tiles_c = jnp.right_shift(tots[c] + (T - 1), jnp.int32(T.bit_length() - 1))
        inc = plsc.cumsum(tiles_c)
        offs.append((inc - tiles_c + carry) * T)
        carry = carry + jnp.sum(tiles_c)

    for c in range(E // L):
        prew = zero16
        for w in range(NW):
            hv = allh[pl.ds(w * E + c * L, L)]
            prew = prew + jnp.where(jnp.int32(w) < wid, hv, zero16)
        bacc = offs[c] + prew
        for l in range(L):
            base[pl.ds(l * E + c * L, L)] = bacc
            bacc = bacc + hist16[pl.ds(l * E + c * L, L)]

    for g in range(CHUNK // L):
        ev = idxv[pl.ds(g * L, L)]
        fl = laneoff + ev
        pos = plsc.load_gather(base, [fl])
        plsc.store_scatter(base, [fl], pos + 1)
        dposv[g >> 2, pl.ds((g & 3) * L, L)] = pos

    pltpu.sync_copy(dposv, dpos_hbm.at[wid])

    bufs = (xbuf0, xbuf1)
    nchunk = CHUNK // 64
    rh = r0
    rn = r1
    s_hist = []
    for c in range(nchunk):
        cur = bufs[c % 2]
        rh.wait()
        s = pltpu.async_copy(cur, xs_hbm.at[dposv.at[c]], wsem)
        s_hist.append(s)
        rh = rn
        if c + 2 < nchunk:
            s_hist[c].wait()
            rn = pltpu.async_copy(
                x_hbm.at[pl.ds(tbase + (c + 2) * 64, 64)], bufs[c % 2], rsem)
    s_hist[nchunk - 2].wait()
    s_hist[nchunk - 1].wait()

    @pl.when(wid == 0)
    def _():
        for c in range(E // L):
            histw[pl.ds(c * L, L)] = tots[c]
        pltpu.sync_copy(histw, counts_hbm)
        s = jnp.float32(0)
        for c in range(E // L):
            f = tots[c].astype(jnp.float32) * jnp.float32(1.0 / S)
            s = s + jnp.sum(f * f)
        balw[...] = jnp.zeros((L,), jnp.float32) + s * jnp.float32(E)
        pltpu.sync_copy(balw, bal_hbm)


def _route(idx_flat, flat, hist):
    return pl.kernel(
        _route_body,
        out_type=[
            jax.ShapeDtypeStruct((E,), jnp.int32),
            jax.ShapeDtypeStruct((NW, CHUNK // 64, 64), jnp.int32),
            jax.ShapeDtypeStruct((PAD, H), jnp.float32),
            jax.ShapeDtypeStruct((L,), jnp.float32),
        ],
        mesh=plsc.VectorSubcoreMesh(core_axis_name="c", subcore_axis_name="s",
                                    num_cores=NC, num_subcores=NS),
        scratch_types=[
            pltpu.VMEM((L * E,), jnp.int32),
            pltpu.VMEM((E,), jnp.int32),
            pltpu.VMEM((NW * E,), jnp.int32),
            pltpu.VMEM((L * E,), jnp.int32),
            pltpu.VMEM((CHUNK,), jnp.int32),
            pltpu.VMEM((CHUNK // 64, 64), jnp.int32),
            pltpu.VMEM((64, H), jnp.float32),
            pltpu.VMEM((64, H), jnp.float32),
            pltpu.VMEM((L,), jnp.float32),
            pltpu.SemaphoreType.DMA,
            pltpu.SemaphoreType.DMA,
        ],
        compiler_params=pltpu.CompilerParams(needs_layout_passes=False),
    )(idx_flat, flat, hist)


def _ffn_body(te_ref, rb_ref, tot_ref, x_ref, w1_ref, w2_ref, o_ref):
    g = pl.program_id(0)

    @pl.when(g < tot_ref[0])
    def _():
        h = jnp.dot(x_ref[...], w1_ref[0], preferred_element_type=jnp.float32)
        h = h * (1.0 / (1.0 + jnp.exp(-h)))
        o_ref[...] = jnp.dot(h, w2_ref[0], preferred_element_type=jnp.float32)


def _ffn(te, rb, tot, xs, W1, W2):
    grid_spec = pltpu.PrefetchScalarGridSpec(
        num_scalar_prefetch=3,
        grid=(G,),
        in_specs=[
            pl.BlockSpec((T, H), lambda g, te, rb, tot: (rb[g], 0)),
            pl.BlockSpec((1, H, I), lambda g, te, rb, tot: (te[g], 0, 0)),
            pl.BlockSpec((1, I, H), lambda g, te, rb, tot: (te[g], 0, 0)),
        ],
        out_specs=pl.BlockSpec((T, H), lambda g, te, rb, tot: (rb[g], 0)),
    )
    return pl.pallas_call(
        _ffn_body,
        grid_spec=grid_spec,
        out_shape=jax.ShapeDtypeStruct((PAD, H), jnp.float32),
    )(te, rb, tot, xs, W1, W2)


def _unsort_body(ys_hbm, dpos_hbm, p_hbm, y_hbm, dposv, pv, ybuf0, ybuf1,
                 gsem, wsem):
    cid = lax.axis_index("c")
    sid = lax.axis_index("s")
    wid = sid * NC + cid
    tbase = wid * CHUNK
    pltpu.sync_copy(dpos_hbm.at[wid], dposv)
    pltpu.sync_copy(p_hbm.at[pl.ds(tbase, CHUNK)], pv)
    bufs = (ybuf0, ybuf1)
    nchunk = CHUNK // 64
    zeroL = jnp.zeros((L,), jnp.int32)
    gh = pltpu.async_copy(ys_hbm.at[dposv.at[0]], ybuf0, gsem)
    gn = pltpu.async_copy(ys_hbm.at[dposv.at[1]], ybuf1, gsem)
    w_hist = []
    for c in range(nchunk):
        cur = bufs[c % 2]
        gh.wait()

        def _scale_row(j, _, cur=cur, c=c):
            pj = plsc.load_gather(pv, [zeroL + (c * 64 + j)])
            for k in range(H // L):
                cur[j, pl.ds(k * L, L)] = cur[j, pl.ds(k * L, L)] * pj
            return _

        lax.fori_loop(0, 64, _scale_row, 0)

        w = pltpu.async_copy(cur, y_hbm.at[pl.ds(tbase + c * 64, 64)], wsem)
        w_hist.append(w)
        gh = gn
        if c + 2 < nchunk:
            w_hist[c].wait()
            gn = pltpu.async_copy(ys_hbm.at[dposv.at[c + 2]], bufs[c % 2], gsem)
    w_hist[nchunk - 2].wait()
    w_hist[nchunk - 1].wait()


def _unsort(ys, dpos, p_flat):
    return pl.kernel(
        _unsort_body,
        out_type=jax.ShapeDtypeStruct((S, H), jnp.float32),
        mesh=plsc.VectorSubcoreMesh(core_axis_name="c", subcore_axis_name="s",
                                    num_cores=NC, num_subcores=NS),
        scratch_types=[
            pltpu.VMEM((CHUNK // 64, 64), jnp.int32),
            pltpu.VMEM((CHUNK,), jnp.float32),
            pltpu.VMEM((64, H), jnp.float32),
            pltpu.VMEM((64, H), jnp.float32),
            pltpu.SemaphoreType.DMA,
            pltpu.SemaphoreType.DMA,
        ],
        compiler_params=pltpu.CompilerParams(needs_layout_passes=False),
    )(ys, dpos, p_flat)


def kernel(x, W_gate, W1, W2):
    flat = x.reshape(S, H)
    idx3, p3, hist = _gate(flat, W_gate)
    idx_flat = idx3.reshape(S)
    counts, dpos, xs, balv = _route(idx_flat, flat, hist.reshape(NW * E))

    tiles = (counts + T - 1) // T
    tb = jnp.cumsum(tiles)
    total = tb[E - 1]
    gid = jnp.arange(G, dtype=jnp.int32)
    te_raw = jnp.searchsorted(tb, gid, side="right").astype(jnp.int32)
    last = jnp.maximum(total - 1, 0)
    te = jnp.where(gid < total, jnp.minimum(te_raw, E - 1), te_raw[last])
    rb = jnp.where(gid < total, gid, last)

    ys = _ffn(te, rb, total.reshape(1), xs, W1, W2)
    yflat = _unsort(ys, dpos, p3.reshape(S))

    return (
        yflat.reshape(1, S, H),
        idx3.reshape(1, S, 1),
        p3.reshape(1, S, 1),
        balv[0],
        counts,
    )


# speedup vs baseline: 1.4694x; 1.0039x over previous
import jax
import jax.numpy as jnp
from jax import lax
from jax.experimental import pallas as pl
from jax.experimental.pallas import tpu as pltpu
from jax.experimental.pallas import tpu_sc as plsc

E = 64
H = 768
I = 3072
S = 8192
T = 256
G = S // T + E
PAD = G * T
NC, NS, L = 2, 16, 16
NW = NC * NS
CHUNK = S // NW
BT = 512


def _gate_body(x_ref, wg_ref, idx_ref, p_ref, hist_ref):
    logits = jnp.dot(x_ref[...], wg_ref[...], preferred_element_type=jnp.float32)
    m = jnp.max(logits, axis=-1, keepdims=True)
    col = lax.broadcasted_iota(jnp.int32, logits.shape, 1)
    idx2 = jnp.argmax(logits, axis=-1, keepdims=True).astype(jnp.int32)
    denom = jnp.sum(jnp.exp(logits - m), axis=-1)
    idx_ref[0, 0, :] = idx2[:, 0]
    p_ref[0, 0, :] = 1.0 / denom
    onehot = jnp.where(col == idx2, 1, 0)
    for c in range(BT // CHUNK):
        hist_ref[0, c, :] = jnp.sum(onehot[c * CHUNK:(c + 1) * CHUNK], axis=0)


def _gate(flat, wg):
    nb = S // BT
    return pl.pallas_call(
        _gate_body,
        grid=(nb,),
        in_specs=[
            pl.BlockSpec((BT, H), lambda b: (b, 0)),
            pl.BlockSpec((H, E), lambda b: (0, 0)),
        ],
        out_specs=[
            pl.BlockSpec((1, 1, BT), lambda b: (b, 0, 0)),
            pl.BlockSpec((1, 1, BT), lambda b: (b, 0, 0)),
            pl.BlockSpec((1, BT // CHUNK, E), lambda b: (b, 0, 0)),
        ],
        out_shape=[
            jax.ShapeDtypeStruct((nb, 1, BT), jnp.int32),
            jax.ShapeDtypeStruct((nb, 1, BT), jnp.float32),
            jax.ShapeDtypeStruct((nb, BT // CHUNK, E), jnp.int32),
        ],
    )(flat, wg)


def _route_body(idx_hbm, x_hbm, hist_hbm,
                counts_hbm, dpos_hbm, xs_hbm, bal_hbm,
                hist16, histw, allh, base, idxv, dposv,
                xbuf0, xbuf1, balw, rsem, wsem):
    cid = lax.axis_index("c")
    sid = lax.axis_index("s")
    wid = sid * NC + cid
    tbase = wid * CHUNK

    pltpu.sync_copy(idx_hbm.at[pl.ds(tbase, CHUNK)], idxv)
    pltpu.sync_copy(hist_hbm, allh)

    r0 = pltpu.async_copy(x_hbm.at[pl.ds(tbase, 64)], xbuf0, rsem)
    r1 = pltpu.async_copy(x_hbm.at[pl.ds(tbase + 64, 64)], xbuf1, rsem)

    zero16 = jnp.zeros((L,), jnp.int32)
    for r in range(L):
        for c in range(E // L):
            hist16[pl.ds(r * E + c * L, L)] = zero16

    lanes = lax.iota(jnp.int32, L)
    laneoff = lanes * E
    for g in range(CHUNK // L):
        ev = idxv[pl.ds(g * L, L)]
        fl = laneoff + ev
        cur = plsc.load_gather(hist16, [fl])
        plsc.store_scatter(hist16, [fl], cur + 1)

    tots = []
    for c in range(E // L):
        acc = zero16
        for w in range(NW):
            acc = acc + allh[pl.ds(w * E + c * L, L)]
        tots.append(acc)
    carry = jnp.int32(0)
    offs = []
    for c in range(E // L):
        tiles_c = jnp.right_shift(tots[c] + (T - 1), jnp.int32(T.bit_length() - 1))
        inc = plsc.cumsum(tiles_c)
        offs.append((inc - tiles_c + carry) * T)
        carry = carry + jnp.sum(tiles_c)

    for c in range(E // L):
        prew = zero16
        for w in range(NW):
            hv = allh[pl.ds(w * E + c * L, L)]
            prew = prew + jnp.where(jnp.int32(w) < wid, hv, zero16)
        bacc = offs[c] + prew
        for l in range(L):
            base[pl.ds(l * E + c * L, L)] = bacc
            bacc = bacc + hist16[pl.ds(l * E + c * L, L)]

    for g in range(CHUNK // L):
        ev = idxv[pl.ds(g * L, L)]
        fl = laneoff + ev
        pos = plsc.load_gather(base, [fl])
        plsc.store_scatter(base, [fl], pos + 1)
        dposv[g >> 2, pl.ds((g & 3) * L, L)] = pos

    pltpu.sync_copy(dposv, dpos_hbm.at[wid])

    bufs = (xbuf0, xbuf1)
    nchunk = CHUNK // 64
    rh = r0
    rn = r1
    s_hist = []
    for c in range(nchunk):
        cur = bufs[c % 2]
        rh.wait()
        s = pltpu.async_copy(cur, xs_hbm.at[dposv.at[c]], wsem)
        s_hist.append(s)
        rh = rn
        if c + 2 < nchunk:
            s_hist[c].wait()
            rn = pltpu.async_copy(
                x_hbm.at[pl.ds(tbase + (c + 2) * 64, 64)], bufs[c % 2], rsem)
    s_hist[nchunk - 2].wait()
    s_hist[nchunk - 1].wait()

    @pl.when(wid == 0)
    def _():
        for c in range(E // L):
            histw[pl.ds(c * L, L)] = tots[c]
        pltpu.sync_copy(histw, counts_hbm)
        s = jnp.float32(0)
        for c in range(E // L):
            f = tots[c].astype(jnp.float32) * jnp.float32(1.0 / S)
            s = s + jnp.sum(f * f)
        balw[...] = jnp.zeros((L,), jnp.float32) + s * jnp.float32(E)
        pltpu.sync_copy(balw, bal_hbm)


def _route(idx_flat, flat, hist):
    return pl.kernel(
        _route_body,
        out_type=[
            jax.ShapeDtypeStruct((E,), jnp.int32),
            jax.ShapeDtypeStruct((NW, CHUNK // 64, 64), jnp.int32),
            jax.ShapeDtypeStruct((PAD, H), jnp.float32),
            jax.ShapeDtypeStruct((L,), jnp.float32),
        ],
        mesh=plsc.VectorSubcoreMesh(core_axis_name="c", subcore_axis_name="s",
                                    num_cores=NC, num_subcores=NS),
        scratch_types=[
            pltpu.VMEM((L * E,), jnp.int32),
            pltpu.VMEM((E,), jnp.int32),
            pltpu.VMEM((NW * E,), jnp.int32),
            pltpu.VMEM((L * E,), jnp.int32),
            pltpu.VMEM((CHUNK,), jnp.int32),
            pltpu.VMEM((CHUNK // 64, 64), jnp.int32),
            pltpu.VMEM((64, H), jnp.float32),
            pltpu.VMEM((64, H), jnp.float32),
            pltpu.VMEM((L,), jnp.float32),
            pltpu.SemaphoreType.DMA,
            pltpu.SemaphoreType.DMA,
        ],
        compiler_params=pltpu.CompilerParams(needs_layout_passes=False),
    )(idx_flat, flat, hist)


def _ffn_body(te_ref, rb_ref, tot_ref, x_ref, w1_ref, w2_ref, o_ref):
    g = pl.program_id(0)

    @pl.when(g < tot_ref[0])
    def _():
        h = jnp.dot(x_ref[...], w1_ref[0], preferred_element_type=jnp.float32)
        h = h * (1.0 / (1.0 + jnp.exp(-h)))
        o_ref[...] = jnp.dot(h, w2_ref[0], preferred_element_type=jnp.float32)


def _ffn(te, rb, tot, xs, W1, W2):
    grid_spec = pltpu.PrefetchScalarGridSpec(
        num_scalar_prefetch=3,
        grid=(G,),
        in_specs=[
            pl.BlockSpec((T, H), lambda g, te, rb, tot: (rb[g], 0)),
            pl.BlockSpec((1, H, I), lambda g, te, rb, tot: (te[g], 0, 0)),
            pl.BlockSpec((1, I, H), lambda g, te, rb, tot: (te[g], 0, 0)),
        ],
        out_specs=pl.BlockSpec((T, H), lambda g, te, rb, tot: (rb[g], 0)),
    )
    return pl.pallas_call(
        _ffn_body,
        grid_spec=grid_spec,
        out_shape=jax.ShapeDtypeStruct((PAD, H), jnp.float32),
    )(te, rb, tot, xs, W1, W2)


def _unsort_body(ys_hbm, dpos_hbm, p_hbm, y_hbm, dposv, pv, ybuf0, ybuf1,
                 gsem, wsem):
    cid = lax.axis_index("c")
    sid = lax.axis_index("s")
    wid = sid * NC + cid
    tbase = wid * CHUNK
    pltpu.sync_copy(dpos_hbm.at[wid], dposv)
    pltpu.sync_copy(p_hbm.at[pl.ds(tbase, CHUNK)], pv)
    bufs = (ybuf0, ybuf1)
    nchunk = CHUNK // 64
    zeroL = jnp.zeros((L,), jnp.int32)
    gh = pltpu.async_copy(ys_hbm.at[dposv.at[0]], ybuf0, gsem)
    gn = pltpu.async_copy(ys_hbm.at[dposv.at[1]], ybuf1, gsem)
    w_hist = []
    for c in range(nchunk):
        cur = bufs[c % 2]
        gh.wait()

        def _scale_row(j, _, cur=cur, c=c):
            pj = plsc.load_gather(pv, [zeroL + (c * 64 + j)])
            for k in range(H // L):
                cur[j, pl.ds(k * L, L)] = cur[j, pl.ds(k * L, L)] * pj
            return _

        lax.fori_loop(0, 64, _scale_row, 0)

        w = pltpu.async_copy(cur, y_hbm.at[pl.ds(tbase + c * 64, 64)], wsem)
        w_hist.append(w)
        gh = gn
        if c + 2 < nchunk:
            w_hist[c].wait()
            gn = pltpu.async_copy(ys_hbm.at[dposv.at[c + 2]], bufs[c % 2], gsem)
    w_hist[nchunk - 2].wait()
    w_hist[nchunk - 1].wait()


def _unsort(ys, dpos, p_flat):
    return pl.kernel(
        _unsort_body,
        out_type=jax.ShapeDtypeStruct((S, H), jnp.float32),
        mesh=plsc.VectorSubcoreMesh(core_axis_name="c", subcore_axis_name="s",
                                    num_cores=NC, num_subcores=NS),
        scratch_types=[
            pltpu.VMEM((CHUNK // 64, 64), jnp.int32),
            pltpu.VMEM((CHUNK,), jnp.float32),
            pltpu.VMEM((64, H), jnp.float32),
            pltpu.VMEM((64, H), jnp.float32),
            pltpu.SemaphoreType.DMA,
            pltpu.SemaphoreType.DMA,
        ],
        compiler_params=pltpu.CompilerParams(needs_layout_passes=False),
    )(ys, dpos, p_flat)


def kernel(x, W_gate, W1, W2):
    flat = x.reshape(S, H)
    idx3, p3, hist = _gate(flat, W_gate)
    idx_flat = idx3.reshape(S)
    counts, dpos, xs, balv = _route(idx_flat, flat, hist.reshape(NW * E))

    tiles = (counts + T - 1) // T
    tb = jnp.cumsum(tiles)
    total = tb[E - 1]
    gid = jnp.arange(G, dtype=jnp.int32)
    te_raw = jnp.searchsorted(tb, gid, side="right").astype(jnp.int32)
    last = jnp.maximum(total - 1, 0)
    te = jnp.where(gid < total, jnp.minimum(te_raw, E - 1), te_raw[last])
    rb = jnp.where(gid < total, gid, last)

    ys = _ffn(te, rb, total.reshape(1), xs, W1, W2)
    yflat = _unsort(ys, dpos, p3.reshape(S))

    return (
        yflat.reshape(1, S, H),
        idx3.reshape(1, S, 1),
        p3.reshape(1, S, 1),
        balv[0],
        counts,
    )


# speedup vs baseline: 1.4931x; 1.0162x over previous
import jax
import jax.numpy as jnp
from jax import lax
from jax.experimental import pallas as pl
from jax.experimental.pallas import tpu as pltpu
from jax.experimental.pallas import tpu_sc as plsc

E = 64
H = 768
I = 3072
S = 8192
T = 192
G = S // T + E
PAD = G * T
NC, NS, L = 2, 16, 16
NW = NC * NS
CHUNK = S // NW
BT = 512


def _gate_body(x_ref, wg_ref, idx_ref, p_ref, hist_ref):
    logits = jnp.dot(x_ref[...], wg_ref[...], preferred_element_type=jnp.float32)
    m = jnp.max(logits, axis=-1, keepdims=True)
    col = lax.broadcasted_iota(jnp.int32, logits.shape, 1)
    idx2 = jnp.argmax(logits, axis=-1, keepdims=True).astype(jnp.int32)
    denom = jnp.sum(jnp.exp(logits - m), axis=-1)
    idx_ref[0, 0, :] = idx2[:, 0]
    p_ref[0, 0, :] = 1.0 / denom
    onehot = jnp.where(col == idx2, 1, 0)
    for c in range(BT // CHUNK):
        hist_ref[0, c, :] = jnp.sum(onehot[c * CHUNK:(c + 1) * CHUNK], axis=0)


def _gate(flat, wg):
    nb = S // BT
    return pl.pallas_call(
        _gate_body,
        grid=(nb,),
        in_specs=[
            pl.BlockSpec((BT, H), lambda b: (b, 0)),
            pl.BlockSpec((H, E), lambda b: (0, 0)),
        ],
        out_specs=[
            pl.BlockSpec((1, 1, BT), lambda b: (b, 0, 0)),
            pl.BlockSpec((1, 1, BT), lambda b: (b, 0, 0)),
            pl.BlockSpec((1, BT // CHUNK, E), lambda b: (b, 0, 0)),
        ],
        out_shape=[
            jax.ShapeDtypeStruct((nb, 1, BT), jnp.int32),
            jax.ShapeDtypeStruct((nb, 1, BT), jnp.float32),
            jax.ShapeDtypeStruct((nb, BT // CHUNK, E), jnp.int32),
        ],
    )(flat, wg)


def _route_body(idx_hbm, x_hbm, hist_hbm,
                counts_hbm, dpos_hbm, xs_hbm, bal_hbm,
                hist16, histw, allh, base, idxv, dposv,
                xbuf0, xbuf1, balw, rsem, wsem):
    cid = lax.axis_index("c")
    sid = lax.axis_index("s")
    wid = sid * NC + cid
    tbase = wid * CHUNK

    pltpu.sync_copy(idx_hbm.at[pl.ds(tbase, CHUNK)], idxv)
    pltpu.sync_copy(hist_hbm, allh)

    r0 = pltpu.async_copy(x_hbm.at[pl.ds(tbase, 64)], xbuf0, rsem)
    r1 = pltpu.async_copy(x_hbm.at[pl.ds(tbase + 64, 64)], xbuf1, rsem)

    zero16 = jnp.zeros((L,), jnp.int32)
    for r in range(L):
        for c in range(E // L):
            hist16[pl.ds(r * E + c * L, L)] = zero16

    lanes = lax.iota(jnp.int32, L)
    laneoff = lanes * E
    for g in range(CHUNK // L):
        ev = idxv[pl.ds(g * L, L)]
        fl = laneoff + ev
        cur = plsc.load_gather(hist16, [fl])
        plsc.store_scatter(hist16, [fl], cur + 1)

    tots = []
    for c in range(E // L):
        acc = zero16
        for w in range(NW):
            acc = acc + allh[pl.ds(w * E + c * L, L)]
        tots.append(acc)
    carry = jnp.int32(0)
    offs = []
    for c in range(E // L):
        tiles_c = (tots[c] + (T - 1)) // T
        inc = plsc.cumsum(tiles_c)
        offs.append((inc - tiles_c + carry) * T)
        carry = carry + jnp.sum(tiles_c)

    for c in range(E // L):
        prew = zero16
        for w in range(NW):
            hv = allh[pl.ds(w * E + c * L, L)]
            prew = prew + jnp.where(jnp.int32(w) < wid, hv, zero16)
        bacc = offs[c] + prew
        for l in range(L):
            base[pl.ds(l * E + c * L, L)] = bacc
            bacc = bacc + hist16[pl.ds(l * E + c * L, L)]

    for g in range(CHUNK // L):
        ev = idxv[pl.ds(g * L, L)]
        fl = laneoff + ev
        pos = plsc.load_gather(base, [fl])
        plsc.store_scatter(base, [fl], pos + 1)
        dposv[g >> 2, pl.ds((g & 3) * L, L)] = pos

    pltpu.sync_copy(dposv, dpos_hbm.at[wid])

    bufs = (xbuf0, xbuf1)
    nchunk = CHUNK // 64
    rh = r0
    rn = r1
    s_hist = []
    for c in range(nchunk):
        cur = bufs[c % 2]
        rh.wait()
        s = pltpu.async_copy(cur, xs_hbm.at[dposv.at[c]], wsem)
        s_hist.append(s)
        rh = rn
        if c + 2 < nchunk:
            s_hist[c].wait()
            rn = pltpu.async_copy(
                x_hbm.at[pl.ds(tbase + (c + 2) * 64, 64)], bufs[c % 2], rsem)
    s_hist[nchunk - 2].wait()
    s_hist[nchunk - 1].wait()

    @pl.when(wid == 0)
    def _():
        for c in range(E // L):
            histw[pl.ds(c * L, L)] = tots[c]
        pltpu.sync_copy(histw, counts_hbm)
        s = jnp.float32(0)
        for c in range(E // L):
            f = tots[c].astype(jnp.float32) * jnp.float32(1.0 / S)
            s = s + jnp.sum(f * f)
        balw[...] = jnp.zeros((L,), jnp.float32) + s * jnp.float32(E)
        pltpu.sync_copy(balw, bal_hbm)


def _route(idx_flat, flat, hist):
    return pl.kernel(
        _route_body,
        out_type=[
            jax.ShapeDtypeStruct((E,), jnp.int32),
            jax.ShapeDtypeStruct((NW, CHUNK // 64, 64), jnp.int32),
            jax.ShapeDtypeStruct((PAD, H), jnp.float32),
            jax.ShapeDtypeStruct((L,), jnp.float32),
        ],
        mesh=plsc.VectorSubcoreMesh(core_axis_name="c", subcore_axis_name="s",
                                    num_cores=NC, num_subcores=NS),
        scratch_types=[
            pltpu.VMEM((L * E,), jnp.int32),
            pltpu.VMEM((E,), jnp.int32),
            pltpu.VMEM((NW * E,), jnp.int32),
            pltpu.VMEM((L * E,), jnp.int32),
            pltpu.VMEM((CHUNK,), jnp.int32),
            pltpu.VMEM((CHUNK // 64, 64), jnp.int32),
            pltpu.VMEM((64, H), jnp.float32),
            pltpu.VMEM((64, H), jnp.float32),
            pltpu.VMEM((L,), jnp.float32),
            pltpu.SemaphoreType.DMA,
            pltpu.SemaphoreType.DMA,
        ],
        compiler_params=pltpu.CompilerParams(needs_layout_passes=False),
    )(idx_flat, flat, hist)


def _ffn_body(te_ref, rb_ref, tot_ref, x_ref, w1_ref, w2_ref, o_ref):
    g = pl.program_id(0)

    @pl.when(g < tot_ref[0])
    def _():
        h = jnp.dot(x_ref[...], w1_ref[0], preferred_element_type=jnp.float32)
        h = h * (1.0 / (1.0 + jnp.exp(-h)))
        o_ref[...] = jnp.dot(h, w2_ref[0], preferred_element_type=jnp.float32)


def _ffn(te, rb, tot, xs, W1, W2):
    grid_spec = pltpu.PrefetchScalarGridSpec(
        num_scalar_prefetch=3,
        grid=(G,),
        in_specs=[
            pl.BlockSpec((T, H), lambda g, te, rb, tot: (rb[g], 0)),
            pl.BlockSpec((1, H, I), lambda g, te, rb, tot: (te[g], 0, 0)),
            pl.BlockSpec((1, I, H), lambda g, te, rb, tot: (te[g], 0, 0)),
        ],
        out_specs=pl.BlockSpec((T, H), lambda g, te, rb, tot: (rb[g], 0)),
    )
    return pl.pallas_call(
        _ffn_body,
        grid_spec=grid_spec,
        out_shape=jax.ShapeDtypeStruct((PAD, H), jnp.float32),
    )(te, rb, tot, xs, W1, W2)


def _unsort_body(ys_hbm, dpos_hbm, p_hbm, y_hbm, dposv, pv, ybuf0, ybuf1,
                 gsem, wsem):
    cid = lax.axis_index("c")
    sid = lax.axis_index("s")
    wid = sid * NC + cid
    tbase = wid * CHUNK
    pltpu.sync_copy(dpos_hbm.at[wid], dposv)
    pltpu.sync_copy(p_hbm.at[pl.ds(tbase, CHUNK)], pv)
    bufs = (ybuf0, ybuf1)
    nchunk = CHUNK // 64
    zeroL = jnp.zeros((L,), jnp.int32)
    gh = pltpu.async_copy(ys_hbm.at[dposv.at[0]], ybuf0, gsem)
    gn = pltpu.async_copy(ys_hbm.at[dposv.at[1]], ybuf1, gsem)
    w_hist = []
    for c in range(nchunk):
        cur = bufs[c % 2]
        gh.wait()

        def _scale_row(j, _, cur=cur, c=c):
            pj = plsc.load_gather(pv, [zeroL + (c * 64 + j)])
            for k in range(H // L):
                cur[j, pl.ds(k * L, L)] = cur[j, pl.ds(k * L, L)] * pj
            return _

        lax.fori_loop(0, 64, _scale_row, 0)

        w = pltpu.async_copy(cur, y_hbm.at[pl.ds(tbase + c * 64, 64)], wsem)
        w_hist.append(w)
        gh = gn
        if c + 2 < nchunk:
            w_hist[c].wait()
            gn = pltpu.async_copy(ys_hbm.at[dposv.at[c + 2]], bufs[c % 2], gsem)
    w_hist[nchunk - 2].wait()
    w_hist[nchunk - 1].wait()


def _unsort(ys, dpos, p_flat):
    return pl.kernel(
        _unsort_body,
        out_type=jax.ShapeDtypeStruct((S, H), jnp.float32),
        mesh=plsc.VectorSubcoreMesh(core_axis_name="c", subcore_axis_name="s",
                                    num_cores=NC, num_subcores=NS),
        scratch_types=[
            pltpu.VMEM((CHUNK // 64, 64), jnp.int32),
            pltpu.VMEM((CHUNK,), jnp.float32),
            pltpu.VMEM((64, H), jnp.float32),
            pltpu.VMEM((64, H), jnp.float32),
            pltpu.SemaphoreType.DMA,
            pltpu.SemaphoreType.DMA,
        ],
        compiler_params=pltpu.CompilerParams(needs_layout_passes=False),
    )(ys, dpos, p_flat)


def kernel(x, W_gate, W1, W2):
    flat = x.reshape(S, H)
    idx3, p3, hist = _gate(flat, W_gate)
    idx_flat = idx3.reshape(S)
    counts, dpos, xs, balv = _route(idx_flat, flat, hist.reshape(NW * E))

    tiles = (counts + T - 1) // T
    tb = jnp.cumsum(tiles)
    total = tb[E - 1]
    gid = jnp.arange(G, dtype=jnp.int32)
    te_raw = jnp.searchsorted(tb, gid, side="right").astype(jnp.int32)
    last = jnp.maximum(total - 1, 0)
    te = jnp.where(gid < total, jnp.minimum(te_raw, E - 1), te_raw[last])
    rb = jnp.where(gid < total, gid, last)

    ys = _ffn(te, rb, total.reshape(1), xs, W1, W2)
    yflat = _unsort(ys, dpos, p3.reshape(S))

    return (
        yflat.reshape(1, S, H),
        idx3.reshape(1, S, 1),
        p3.reshape(1, S, 1),
        balv[0],
        counts,
    )
